# baseline (speedup 1.0000x reference)
import functools

import jax
import jax.numpy as jnp
from jax import lax
from jax.experimental import pallas as pl
from jax.experimental.pallas import tpu as pltpu
from jax.experimental.pallas import tpu_sc as plsc

_SMOOTHING = 0.1
_IGNORE_INDEX = -100

_NW = 32
_L = 16


def _sc_body(pred_hbm, gidx_hbm, out_t_hbm, out_g_hbm,
             buf, idx_v, gv, ov, og, sem0, sem1, semg, semi,
             *, span, chunk, t_dim):
    nchunk = span // chunk
    c = lax.axis_index("c")
    s = lax.axis_index("s")
    wid = s * 2 + c
    base = wid * span

    pltpu.async_copy(gidx_hbm.at[wid], idx_v, semi).wait()
    gcp = pltpu.async_copy(pred_hbm.at[idx_v], gv, semg)

    cp0 = pltpu.async_copy(pred_hbm.at[pl.ds(base, chunk)], buf.at[0], sem0)
    cp1 = pltpu.async_copy(pred_hbm.at[pl.ds(base + chunk, chunk)],
                           buf.at[1], sem1)

    half = t_dim // 2

    def consume(bslot, acc0, acc1):
        def inner(i, accs):
            a0, a1 = accs
            off = i * t_dim
            v0 = bslot[pl.ds(off, _L)]
            v1 = bslot[pl.ds(off + half, _L)]
            return a0 + v0, a1 + v1
        return lax.fori_loop(0, chunk // t_dim, inner, (acc0, acc1))

    def outer(k, accs):
        a0, a1 = accs
        cp0.wait()
        a0, a1 = consume(buf.at[0], a0, a1)
        pltpu.async_copy(
            pred_hbm.at[pl.ds(base + (2 * k + 2) * chunk, chunk)],
            buf.at[0], sem0)
        cp1.wait()
        a0, a1 = consume(buf.at[1], a0, a1)
        pltpu.async_copy(
            pred_hbm.at[pl.ds(base + (2 * k + 3) * chunk, chunk)],
            buf.at[1], sem1)
        return a0, a1

    zero = jnp.zeros((_L,), jnp.float32)
    acc0, acc1 = lax.fori_loop(0, nchunk // 2 - 1, outer, (zero, zero))

    cp0.wait()
    acc0, acc1 = consume(buf.at[0], acc0, acc1)
    cp1.wait()
    acc0, acc1 = consume(buf.at[1], acc0, acc1)

    ov[0] = acc0
    ov[1] = acc1
    pltpu.sync_copy(ov, out_t_hbm.at[wid])

    gcp.wait()
    og[...] = gv[...]
    pltpu.sync_copy(og, out_g_hbm.at[wid])


def kernel(pred, target):
    B, C, T = pred.shape
    n_words = B * C * T
    span = n_words // _NW
    chunk = 16000
    assert span % (2 * chunk) == 0 and chunk % T == 0
    assert (C * T) % span == 0 or span % (C * T) == 0

    pred_flat = pred.reshape(n_words)

    tgt = target.astype(jnp.int32)
    mask = tgt != _IGNORE_INDEX
    safe_tgt = jnp.where(mask, tgt, 0)
    t_idx = jnp.arange(T, dtype=jnp.int32)[None, :]
    b_idx = jnp.arange(B, dtype=jnp.int32)[:, None]
    flat_idx = (b_idx * (C * T) + safe_tgt * T + t_idx).reshape(_NW, _L)

    mesh = plsc.VectorSubcoreMesh(core_axis_name="c", subcore_axis_name="s")
    body = functools.partial(_sc_body, span=span, chunk=chunk, t_dim=T)
    out_t, out_g = pl.kernel(
        body,
        mesh=mesh,
        out_type=[
            jax.ShapeDtypeStruct((_NW, 2, _L), jnp.float32),
            jax.ShapeDtypeStruct((_NW, _L), jnp.float32),
        ],
        scratch_types=[
            pltpu.VMEM((2, chunk), jnp.float32),
            pltpu.VMEM((_L,), jnp.int32),
            pltpu.VMEM((_L,), jnp.float32),
            pltpu.VMEM((2, _L), jnp.float32),
            pltpu.VMEM((_L,), jnp.float32),
            pltpu.SemaphoreType.DMA,
            pltpu.SemaphoreType.DMA,
            pltpu.SemaphoreType.DMA,
            pltpu.SemaphoreType.DMA,
        ],
    )(pred_flat, flat_idx)

    maskf = mask.astype(jnp.float32)
    mask_w = jnp.repeat(maskf, _NW // B, axis=0).reshape(_NW, 2, _L)
    tot = jnp.sum(out_t * mask_w)
    gth = jnp.sum(out_g * maskf.reshape(_NW, _L))

    fill = _SMOOTHING / (C - 1)
    conf = 1.0 - _SMOOTHING
    cnt = jnp.sum(mask)
    denom = jnp.maximum(cnt, 1).astype(pred.dtype)
    loss = -(fill * tot + (conf - fill) * gth) / denom
    return jnp.where(cnt > 0, loss, jnp.zeros((), dtype=pred.dtype))


# speedup vs baseline: 5.7072x; 5.7072x over previous
import functools

import jax
import jax.numpy as jnp
from jax.experimental import pallas as pl
from jax.experimental.pallas import tpu as pltpu

_SMOOTHING = 0.1
_IGNORE_INDEX = -100
_LANES = 128


def _loss_body(pred_ref, tgt_ref, sum_ref, gth_ref, *, bc, c_dim, t_dim):
    jc = pl.program_id(1)

    @pl.when(jc == 0)
    def _init():
        sum_ref[...] = jnp.zeros_like(sum_ref)
        gth_ref[...] = jnp.zeros_like(gth_ref)

    data = pred_ref[0]
    tgt = tgt_ref[0]
    lane = jax.lax.broadcasted_iota(jnp.int32, (t_dim, _LANES), 1)

    s_acc = sum_ref[0]
    g_acc = gth_ref[0]
    base = jc * bc
    for i in range(bc // _LANES):
        sl = data[:, i * _LANES:(i + 1) * _LANES]
        cls = (base + i * _LANES) + lane
        slm = jnp.where(cls < c_dim, sl, 0.0)
        s_acc = s_acc + slm
        g_acc = g_acc + jnp.where(cls == tgt, slm, 0.0)
    sum_ref[0] = s_acc
    gth_ref[0] = g_acc


def kernel(pred, target):
    B, C, T = pred.shape
    BC = 8192
    ncb = -(-C // BC)

    pred_t = jnp.transpose(pred, (0, 2, 1))
    tgt3 = target.astype(jnp.int32).reshape(B, T, 1)

    body = functools.partial(_loss_body, bc=BC, c_dim=C, t_dim=T)
    out_s, out_g = pl.pallas_call(
        body,
        grid=(B, ncb),
        in_specs=[
            pl.BlockSpec((1, T, BC), lambda b, jc: (b, 0, jc)),
            pl.BlockSpec((1, T, 1), lambda b, jc: (b, 0, 0)),
        ],
        out_specs=[
            pl.BlockSpec((1, T, _LANES), lambda b, jc: (b, 0, 0)),
            pl.BlockSpec((1, T, _LANES), lambda b, jc: (b, 0, 0)),
        ],
        out_shape=[
            jax.ShapeDtypeStruct((B, T, _LANES), jnp.float32),
            jax.ShapeDtypeStruct((B, T, _LANES), jnp.float32),
        ],
        compiler_params=pltpu.CompilerParams(
            dimension_semantics=("arbitrary", "arbitrary"),
        ),
    )(pred_t, tgt3)

    mask = target != _IGNORE_INDEX
    maskf = mask.astype(jnp.float32)
    tot = jnp.sum(jnp.sum(out_s, axis=-1) * maskf)
    gth = jnp.sum(jnp.sum(out_g, axis=-1) * maskf)

    fill = _SMOOTHING / (C - 1)
    conf = 1.0 - _SMOOTHING
    cnt = jnp.sum(mask)
    denom = jnp.maximum(cnt, 1).astype(pred.dtype)
    loss = -(fill * tot + (conf - fill) * gth) / denom
    return jnp.where(cnt > 0, loss, jnp.zeros((), dtype=pred.dtype))


# speedup vs baseline: 8.7591x; 1.5348x over previous
import functools

import jax
import jax.numpy as jnp
from jax.experimental import pallas as pl
from jax.experimental.pallas import tpu as pltpu

_SMOOTHING = 0.1
_IGNORE_INDEX = -100
_LANES = 128


def _loss_body(pred_ref, tgt_ref, sum_ref, gth_ref, *, bc, c_dim, t_dim):
    jc = pl.program_id(1)

    @pl.when(jc == 0)
    def _init():
        sum_ref[...] = jnp.zeros_like(sum_ref)
        gth_ref[...] = jnp.zeros_like(gth_ref)

    data = pred_ref[0]
    tgt = tgt_ref[0]
    lane = jax.lax.broadcasted_iota(jnp.int32, (t_dim, _LANES), 1)

    s_acc = sum_ref[0]
    g_acc = gth_ref[0]
    base = jc * bc
    for i in range(bc // _LANES):
        sl = data[:, i * _LANES:(i + 1) * _LANES]
        cls = (base + i * _LANES) + lane
        slm = jnp.where(cls < c_dim, sl, 0.0)
        s_acc = s_acc + slm
        g_acc = g_acc + jnp.where(cls == tgt, slm, 0.0)
    sum_ref[0] = s_acc
    gth_ref[0] = g_acc


def kernel(pred, target):
    B, C, T = pred.shape
    BC = 32768
    ncb = -(-C // BC)

    pred_t = jnp.transpose(pred, (0, 2, 1))
    tgt3 = target.astype(jnp.int32).reshape(B, T, 1)

    body = functools.partial(_loss_body, bc=BC, c_dim=C, t_dim=T)
    out_s, out_g = pl.pallas_call(
        body,
        grid=(B, ncb),
        in_specs=[
            pl.BlockSpec((1, T, BC), lambda b, jc: (b, 0, jc)),
            pl.BlockSpec((1, T, 1), lambda b, jc: (b, 0, 0)),
        ],
        out_specs=[
            pl.BlockSpec((1, T, _LANES), lambda b, jc: (b, 0, 0)),
            pl.BlockSpec((1, T, _LANES), lambda b, jc: (b, 0, 0)),
        ],
        out_shape=[
            jax.ShapeDtypeStruct((B, T, _LANES), jnp.float32),
            jax.ShapeDtypeStruct((B, T, _LANES), jnp.float32),
        ],
        compiler_params=pltpu.CompilerParams(
            dimension_semantics=("arbitrary", "arbitrary"),
        ),
    )(pred_t, tgt3)

    mask = target != _IGNORE_INDEX
    maskf = mask.astype(jnp.float32)
    tot = jnp.sum(jnp.sum(out_s, axis=-1) * maskf)
    gth = jnp.sum(jnp.sum(out_g, axis=-1) * maskf)

    fill = _SMOOTHING / (C - 1)
    conf = 1.0 - _SMOOTHING
    cnt = jnp.sum(mask)
    denom = jnp.maximum(cnt, 1).astype(pred.dtype)
    loss = -(fill * tot + (conf - fill) * gth) / denom
    return jnp.where(cnt > 0, loss, jnp.zeros((), dtype=pred.dtype))
